# Initial kernel scaffold; baseline (speedup 1.0000x reference)
import jax
import jax.numpy as jnp
from jax.experimental import pallas as pl


def kernel(x, spd, dist_table, W, b):
    raise NotImplementedError("write your pallas kernel here")


# speedup vs baseline: 5.7103x; 5.7103x over previous
import functools

import jax
import jax.numpy as jnp
from jax.experimental import pallas as pl
from jax.experimental.pallas import tpu as pltpu


def _pick_block(n):
    for blk in (2000, 1000, 500, 200, 100, 40, 8):
        if n % blk == 0:
            return blk
    return n


def _fused_body(x_ref, spd_ref, tab_ref, w_ref, b_ref, out_ref, *, rows):
    x = x_ref[...]
    h = jnp.dot(x, w_ref[...], preferred_element_type=jnp.float32) + b_ref[...]
    spd = spd_ref[...]
    tab = tab_ref[...]
    blk = x.shape[0]
    iot = jax.lax.broadcasted_iota(jnp.int32, (blk, rows), 1)
    oh0 = (spd[:, 0:1] == iot).astype(jnp.float32)
    oh1 = (spd[:, 1:2] == iot).astype(jnp.float32)
    pe0 = jnp.dot(oh0, tab, preferred_element_type=jnp.float32)
    pe1 = jnp.dot(oh1, tab, preferred_element_type=jnp.float32)
    out_ref[...] = jnp.concatenate([h, pe0, pe1], axis=1)


def kernel(x, spd, dist_table, W, b):
    n, dim_in = x.shape
    dim_h = W.shape[1]
    rows, half_pe = dist_table.shape
    dim_out = dim_h + 2 * half_pe
    blk = _pick_block(n)
    grid = (n // blk,)
    b2 = b.reshape(1, dim_h)
    return pl.pallas_call(
        functools.partial(_fused_body, rows=rows),
        grid=grid,
        in_specs=[
            pl.BlockSpec((blk, dim_in), lambda i: (i, 0)),
            pl.BlockSpec((blk, 2), lambda i: (i, 0)),
            pl.BlockSpec((rows, half_pe), lambda i: (0, 0)),
            pl.BlockSpec((dim_in, dim_h), lambda i: (0, 0)),
            pl.BlockSpec((1, dim_h), lambda i: (0, 0)),
        ],
        out_specs=pl.BlockSpec((blk, dim_out), lambda i: (i, 0)),
        out_shape=jax.ShapeDtypeStruct((n, dim_out), jnp.float32),
        compiler_params=pltpu.CompilerParams(
            dimension_semantics=("parallel",),
        ),
    )(x, spd, dist_table, W, b2)
